# baseline (speedup 1.0000x reference)
import functools

import jax
import jax.numpy as jnp
from jax import lax
from jax.experimental import pallas as pl
from jax.experimental.pallas import tpu as pltpu
from jax.experimental.pallas import tpu_sc as plsc

_DIM = 1024
_SCALE = _DIM ** (-0.5)
_LANES = 16
_NSEG = 8
_SEG = _DIM // _NSEG

_mesh = plsc.VectorSubcoreMesh(core_axis_name="c", subcore_axis_name="s", num_cores=1)


@functools.partial(
    pl.kernel,
    mesh=_mesh,
    out_type=jax.ShapeDtypeStruct((_NSEG, _SEG), jnp.float32),
    scratch_types=[
        pltpu.VMEM((_LANES,), jnp.int32),
        pltpu.VMEM((_LANES,), jnp.int32),
        pltpu.VMEM((1, _SEG), jnp.float32),
        pltpu.SemaphoreType.DMA,
    ],
)
def _lookup(idx_hbm, emb_hbm, out_hbm, idx_v, idx1_v, seg_v, sem):
    sid = lax.axis_index("s")

    @pl.when(sid < _NSEG)
    def _():
        pltpu.sync_copy(idx_hbm, idx_v)
        idx1_v[...] = idx_v[...] * _NSEG + sid
        pltpu.async_copy(emb_hbm.at[idx1_v.at[pl.ds(0, 1)]], seg_v, sem).wait()
        for i in range(_SEG // _LANES):
            sl = pl.ds(i * _LANES, _LANES)
            seg_v[0, sl] = seg_v[0, sl] * _SCALE
        pltpu.sync_copy(seg_v, out_hbm.at[pl.ds(sid, 1)])


def kernel(x, agent, emb):
    del x
    idx16 = jnp.full((_LANES,), agent, dtype=jnp.int32)
    out = _lookup(idx16, emb.reshape(_NSEG * emb.shape[0], _SEG))
    return out.reshape(1, _DIM)


# speedup vs baseline: 1.0411x; 1.0411x over previous
import functools

import jax
import jax.numpy as jnp
from jax.experimental import pallas as pl
from jax.experimental.pallas import tpu as pltpu
from jax.experimental.pallas import tpu_sc as plsc

_DIM = 1024
_SCALE = _DIM ** (-0.5)
_LANES = 16

_mesh = plsc.VectorSubcoreMesh(
    core_axis_name="c", subcore_axis_name="s", num_cores=1, num_subcores=1
)


@functools.partial(
    pl.kernel,
    mesh=_mesh,
    out_type=jax.ShapeDtypeStruct((1, _DIM), jnp.float32),
    scratch_types=[
        pltpu.VMEM((1,), jnp.int32),
        pltpu.VMEM((1, _DIM), jnp.float32),
        pltpu.SemaphoreType.DMA,
    ],
)
def _lookup(idx_hbm, emb_hbm, out_hbm, idx_v, row_v, sem):
    pltpu.sync_copy(idx_hbm, idx_v)
    pltpu.async_copy(emb_hbm.at[idx_v], row_v, sem).wait()
    for i in range(_DIM // _LANES):
        sl = pl.ds(i * _LANES, _LANES)
        row_v[0, sl] = row_v[0, sl] * _SCALE
    pltpu.sync_copy(row_v, out_hbm)


def kernel(x, agent, emb):
    del x
    idx = jnp.asarray(agent, dtype=jnp.int32).reshape((1,))
    return _lookup(idx, emb)
